# Initial kernel scaffold; baseline (speedup 1.0000x reference)
import jax
import jax.numpy as jnp
from jax.experimental import pallas as pl


def kernel(h, edge_attr, edges, node_mask, edge_mask, W, b, gamma, beta, aW1, ab1, aW2, ab2):
    raise NotImplementedError("write your pallas kernel here")


# speedup vs baseline: 2.9752x; 2.9752x over previous
import functools

import jax
import jax.numpy as jnp
from jax import lax
from jax.experimental import pallas as pl
from jax.experimental.pallas import tpu as pltpu
from jax.experimental.pallas import tpu_sc as plsc

N = 10000
E = 320000
D = 128
EPS = 1e-7

NC = 2
NS = 16
NW = NC * NS
CHUNK = 80
EPT = E // NW
NCHUNK = EPT // CHUNK
RPT = 632
NPAD = NS * RPT

BE = 2000


def _lane_mask(shape):
    return lax.broadcasted_iota(jnp.int32, shape, len(shape) - 1) >= 1


def _arccosh(z):
    return jnp.log(z + jnp.sqrt((z - 1.0) * (z + 1.0)))


def _cosh_sinh(t):
    e = jnp.exp(t)
    ei = 1.0 / e
    return 0.5 * (e + ei), 0.5 * (e - ei)


def _sigmoid(t):
    return 1.0 / (1.0 + jnp.exp(-t))


def _node_linear_body(h_ref, w_ref, b_ref, x_ref):
    h = h_ref[...]
    sp = _lane_mask(h.shape)
    h0 = h[:, 0:1]
    d = _arccosh(jnp.maximum(h0, 1.0 + EPS))
    n = jnp.maximum(
        jnp.sqrt(jnp.sum(jnp.where(sp, h * h, 0.0), axis=1, keepdims=True)), 1e-8)
    lm = jnp.where(sp, h * (d / n), 0.0)
    xt = jnp.dot(lm, w_ref[...], preferred_element_type=jnp.float32)
    n2 = jnp.maximum(
        jnp.sqrt(jnp.sum(jnp.where(sp, xt * xt, 0.0), axis=1, keepdims=True)), 1e-8)
    c2, s2 = _cosh_sinh(n2)
    x = jnp.where(sp, xt * (s2 / n2), c2)
    bf = b_ref[...]
    ip = jnp.sum(jnp.where(sp, x * bf, 0.0), axis=1, keepdims=True)
    coef = ip / (1.0 + x[:, 0:1])
    bias = jnp.where(sp, bf + coef * x, ip)
    linn = jnp.sum(jnp.where(sp, bias * bias, -(bias * bias)), axis=1, keepdims=True)
    nrm = jnp.sqrt(jnp.maximum(linn, 1e-12))
    c3, s3 = _cosh_sinh(nrm)
    x_ref[...] = c3 * x + (s3 / nrm) * bias


def _stage_a(h, w, b):
    return pl.pallas_call(
        _node_linear_body,
        out_shape=jax.ShapeDtypeStruct((N, D), jnp.float32),
    )(h, w, b)


def _sc_gather_body(x_hbm, row_hbm, col_hbm, xr_hbm, xc_hbm,
                    ridx_v, cidx_v, rbuf_v, cbuf_v, rsem, csem):
    wid = lax.axis_index("s") * NC + lax.axis_index("c")
    base = wid * EPT

    def body(j, carry):
        pltpu.sync_copy(row_hbm.at[pl.ds(base + j * CHUNK, CHUNK)], ridx_v)
        pltpu.sync_copy(col_hbm.at[pl.ds(base + j * CHUNK, CHUNK)], cidx_v)
        cpr = pltpu.async_copy(x_hbm.at[ridx_v], rbuf_v, rsem)
        cpc = pltpu.async_copy(x_hbm.at[cidx_v], cbuf_v, csem)
        cpr.wait()
        cpc.wait()
        off = base + j * CHUNK
        pltpu.sync_copy(rbuf_v, xr_hbm.at[pl.ds(off, CHUNK)])
        pltpu.sync_copy(cbuf_v, xc_hbm.at[pl.ds(off, CHUNK)])
        return carry

    lax.fori_loop(0, NCHUNK, body, 0)


def _sc_gather(x, row1, col1):
    fn = functools.partial(
        pl.kernel,
        out_type=[jax.ShapeDtypeStruct((E, D), jnp.float32),
                  jax.ShapeDtypeStruct((E, D), jnp.float32)],
        mesh=plsc.VectorSubcoreMesh(core_axis_name="c", subcore_axis_name="s"),
        scratch_types=[
            pltpu.VMEM((CHUNK,), jnp.int32),
            pltpu.VMEM((CHUNK,), jnp.int32),
            pltpu.VMEM((CHUNK, D), jnp.float32),
            pltpu.VMEM((CHUNK, D), jnp.float32),
            pltpu.SemaphoreType.DMA,
            pltpu.SemaphoreType.DMA,
        ],
    )(_sc_gather_body)
    return fn(x, row1, col1)


def _edge_body(xr_ref, xc_ref, ea_ref, em_ref, w1r_ref, w1c_ref, w1e_ref,
               b1_ref, w2_ref, b2_ref, agg_ref):
    gr = xr_ref[...]
    gc = xc_ref[...]
    sp = _lane_mask(gr.shape)

    def tan0(g):
        d = _arccosh(jnp.maximum(g[:, 0:1], 1.0 + EPS))
        n = jnp.maximum(
            jnp.sqrt(jnp.sum(jnp.where(sp, g * g, 0.0), axis=1, keepdims=True)),
            1e-8)
        return jnp.where(sp, g * (d / n), 0.0)

    s = jnp.dot(tan0(gr), w1r_ref[...], preferred_element_type=jnp.float32)
    s = s + jnp.dot(tan0(gc), w1c_ref[...], preferred_element_type=jnp.float32)
    s = s + jnp.dot(ea_ref[...], w1e_ref[...], preferred_element_type=jnp.float32)
    s = s + b1_ref[...]
    hid = s * _sigmoid(s)
    logit = jnp.sum(hid * w2_ref[...], axis=1, keepdims=True) + b2_ref[...][:, 0:1]
    att = _sigmoid(logit) * em_ref[...]
    alpha = jnp.maximum(
        jnp.sum(jnp.where(sp, -(gr * gc), gr * gc), axis=1, keepdims=True),
        1.0 + EPS)
    dd = _arccosh(alpha)
    denom = jnp.sqrt(jnp.maximum((alpha - 1.0) * (alpha + 1.0), 1e-12))
    agg_ref[...] = (att * dd / denom) * (gc - alpha * gr)


def _edge_stage(xr, xc, ea_pad, edge_mask, w1r, w1c, w1e, b1, w2row, b2row):
    grid = (E // BE,)
    return pl.pallas_call(
        _edge_body,
        grid=grid,
        in_specs=[
            pl.BlockSpec((BE, D), lambda i: (i, 0)),
            pl.BlockSpec((BE, D), lambda i: (i, 0)),
            pl.BlockSpec((BE, 8), lambda i: (i, 0)),
            pl.BlockSpec((BE, 1), lambda i: (i, 0)),
            pl.BlockSpec((D, D), lambda i: (0, 0)),
            pl.BlockSpec((D, D), lambda i: (0, 0)),
            pl.BlockSpec((8, D), lambda i: (0, 0)),
            pl.BlockSpec((1, D), lambda i: (0, 0)),
            pl.BlockSpec((1, D), lambda i: (0, 0)),
            pl.BlockSpec((1, D), lambda i: (0, 0)),
        ],
        out_specs=pl.BlockSpec((BE, D), lambda i: (i, 0)),
        out_shape=jax.ShapeDtypeStruct((E, D), jnp.float32),
    )(xr, xc, ea_pad, edge_mask, w1r, w1c, w1e, b1, w2row, b2row)


def _sc_scatter_body(agg_hbm, row_hbm, z_hbm, out_hbm, idx_v, val_v, acc_sh, sem):
    c = lax.axis_index("c")
    s = lax.axis_index("s")
    wid = s * NC + c
    rbase = s * RPT
    pltpu.sync_copy(z_hbm, acc_sh.at[pl.ds(rbase, RPT)])
    plsc.subcore_barrier()

    def body(j, carry):
        pltpu.sync_copy(row_hbm.at[pl.ds(wid * EPT + j * CHUNK, CHUNK)], idx_v)
        pltpu.sync_copy(agg_hbm.at[pl.ds(wid * EPT + j * CHUNK, CHUNK)], val_v)
        pltpu.sync_copy(val_v, acc_sh.at[idx_v], add=True)
        return carry

    lax.fori_loop(0, NCHUNK, body, 0)
    plsc.subcore_barrier()
    pltpu.sync_copy(acc_sh.at[pl.ds(rbase, RPT)],
                    out_hbm.at[c, pl.ds(rbase, RPT)])
    del sem


def _sc_scatter(agg, row1, zeros_tile):
    fn = functools.partial(
        pl.kernel,
        out_type=jax.ShapeDtypeStruct((NC, NPAD, D), jnp.float32),
        mesh=plsc.VectorSubcoreMesh(core_axis_name="c", subcore_axis_name="s"),
        scratch_types=[
            pltpu.VMEM((CHUNK,), jnp.int32),
            pltpu.VMEM((CHUNK, D), jnp.float32),
            pltpu.VMEM_SHARED((NPAD, D), jnp.float32),
            pltpu.SemaphoreType.DMA,
        ],
    )(_sc_scatter_body)
    return fn(agg, row1, zeros_tile)


def _final_body(x_ref, p0_ref, p1_ref, gam_ref, bet_ref, out_ref):
    x = x_ref[...]
    sp = _lane_mask(x.shape)
    agg = p0_ref[...] + p1_ref[...]
    li = jnp.sum(jnp.where(sp, x * agg, -(x * agg)), axis=1, keepdims=True)
    u = agg + li * x
    linn = jnp.sum(jnp.where(sp, u * u, -(u * u)), axis=1, keepdims=True)
    nrm = jnp.sqrt(jnp.maximum(linn, 1e-12))
    ch, sh = _cosh_sinh(nrm)
    x2 = ch * x + (sh / nrm) * u
    d = _arccosh(jnp.maximum(x2[:, 0:1], 1.0 + EPS))
    n = jnp.maximum(
        jnp.sqrt(jnp.sum(jnp.where(sp, x2 * x2, 0.0), axis=1, keepdims=True)), 1e-8)
    ht = jnp.where(sp, x2 * (d / n), 0.0)
    mu = jnp.sum(ht, axis=1, keepdims=True) / 127.0
    dsp = jnp.where(sp, ht - mu, 0.0)
    var = jnp.sum(dsp * dsp, axis=1, keepdims=True) / 127.0
    spn = dsp / jnp.sqrt(var + 1e-5) * gam_ref[...] + bet_ref[...]
    n3 = jnp.maximum(
        jnp.sqrt(jnp.sum(jnp.where(sp, spn * spn, 0.0), axis=1, keepdims=True)),
        1e-8)
    c3, s3 = _cosh_sinh(n3)
    x3 = jnp.where(sp, spn * (s3 / n3), c3)
    d4 = _arccosh(jnp.maximum(x3[:, 0:1], 1.0 + EPS))
    n4 = jnp.maximum(
        jnp.sqrt(jnp.sum(jnp.where(sp, x3 * x3, 0.0), axis=1, keepdims=True)), 1e-8)
    r = jnp.maximum(jnp.where(sp, x3 * (d4 / n4), 0.0), 0.0)
    n5 = jnp.maximum(jnp.sqrt(jnp.sum(r * r, axis=1, keepdims=True)), 1e-8)
    c5, s5 = _cosh_sinh(n5)
    out_ref[...] = jnp.where(sp, r * (s5 / n5), c5)


def _final_stage(x, p0, p1, gam, bet):
    return pl.pallas_call(
        _final_body,
        out_shape=jax.ShapeDtypeStruct((N, D), jnp.float32),
    )(x, p0, p1, gam, bet)


def kernel(h, edge_attr, edges, node_mask, edge_mask, W, b, gamma, beta,
           aW1, ab1, aW2, ab2):
    del node_mask
    row = edges[0].astype(jnp.int32)
    col = edges[1].astype(jnp.int32)

    x = _stage_a(h, W, b)
    xr, xc = _sc_gather(x, row, col)

    ea_pad = jnp.concatenate(
        [edge_attr, jnp.zeros((E, 8 - edge_attr.shape[1]), jnp.float32)], axis=1)
    w1r = aW1[:D]
    w1c = aW1[D:2 * D]
    w1e = jnp.concatenate(
        [aW1[2 * D:], jnp.zeros((8 - (aW1.shape[0] - 2 * D), D), jnp.float32)],
        axis=0)
    b1 = ab1.reshape(1, D)
    w2row = aW2.reshape(1, D)
    b2row = jnp.broadcast_to(ab2.reshape(1, 1), (1, D))

    agg = _edge_stage(xr, xc, ea_pad, edge_mask, w1r, w1c, w1e, b1, w2row, b2row)

    zeros_tile = jnp.zeros((RPT, D), jnp.float32)
    partials = _sc_scatter(agg, row, zeros_tile)

    gam = jnp.concatenate([jnp.ones((1,), jnp.float32), gamma]).reshape(1, D)
    bet = jnp.concatenate([jnp.zeros((1,), jnp.float32), beta]).reshape(1, D)
    return _final_stage(x, partials[0, :N], partials[1, :N], gam, bet)


# speedup vs baseline: 4.0267x; 1.3534x over previous
import functools

import jax
import jax.numpy as jnp
from jax import lax
from jax.experimental import pallas as pl
from jax.experimental.pallas import tpu as pltpu
from jax.experimental.pallas import tpu_sc as plsc

N = 10000
E = 320000
D = 128
EPS = 1e-7

NC = 2
NS = 16
NW = NC * NS
RPT = 632
NPAD = NS * RPT

BE = 2000
NSPLIT = 2
EH = E // NSPLIT
EPT_H = EH // NW
CHUNK = 40
NCHUNK_H = EPT_H // CHUNK


def _lane_mask(shape):
    return lax.broadcasted_iota(jnp.int32, shape, len(shape) - 1) >= 1


def _arccosh(z):
    return jnp.log(z + jnp.sqrt((z - 1.0) * (z + 1.0)))


def _cosh_sinh(t):
    e = jnp.exp(t)
    ei = 1.0 / e
    return 0.5 * (e + ei), 0.5 * (e - ei)


def _node_linear_body(h_ref, w_ref, b_ref, x_ref):
    h = h_ref[...]
    sp = _lane_mask(h.shape)
    h0 = h[:, 0:1]
    d = _arccosh(jnp.maximum(h0, 1.0 + EPS))
    n = jnp.maximum(
        jnp.sqrt(jnp.sum(jnp.where(sp, h * h, 0.0), axis=1, keepdims=True)), 1e-8)
    lm = jnp.where(sp, h * (d / n), 0.0)
    xt = jnp.dot(lm, w_ref[...], preferred_element_type=jnp.float32)
    n2 = jnp.maximum(
        jnp.sqrt(jnp.sum(jnp.where(sp, xt * xt, 0.0), axis=1, keepdims=True)), 1e-8)
    c2, s2 = _cosh_sinh(n2)
    x = jnp.where(sp, xt * (s2 / n2), c2)
    bf = b_ref[...]
    ip = jnp.sum(jnp.where(sp, x * bf, 0.0), axis=1, keepdims=True)
    coef = ip / (1.0 + x[:, 0:1])
    bias = jnp.where(sp, bf + coef * x, ip)
    linn = jnp.sum(jnp.where(sp, bias * bias, -(bias * bias)), axis=1, keepdims=True)
    nrm = jnp.sqrt(jnp.maximum(linn, 1e-12))
    c3, s3 = _cosh_sinh(nrm)
    x_ref[...] = c3 * x + (s3 / nrm) * bias


def _stage_a(h, w, b):
    return pl.pallas_call(
        _node_linear_body,
        out_shape=jax.ShapeDtypeStruct((N, D), jnp.float32),
    )(h, w, b)


def _sc_gather(x, row1, col1):
    e_tot = row1.shape[0]

    def body(x_hbm, row_hbm, col_hbm, xr_hbm, xc_hbm,
             ridx_a, cidx_a, rbuf_a, cbuf_a,
             ridx_b, cidx_b, rbuf_b, cbuf_b, sem_a, sem_b):
        wid = lax.axis_index("s") * NC + lax.axis_index("c")
        base = wid * EPT_H

        def issue(c, ridx, cidx, rbuf, cbuf, sem):
            off = base + c * CHUNK
            pltpu.sync_copy(row_hbm.at[pl.ds(off, CHUNK)], ridx)
            pltpu.sync_copy(col_hbm.at[pl.ds(off, CHUNK)], cidx)
            pltpu.async_copy(x_hbm.at[ridx], rbuf, sem)
            pltpu.async_copy(x_hbm.at[cidx], cbuf, sem)

        def drain_store(c, ridx, cidx, rbuf, cbuf, sem):
            pltpu.make_async_copy(x_hbm.at[ridx], rbuf, sem).wait()
            pltpu.make_async_copy(x_hbm.at[cidx], cbuf, sem).wait()
            off = base + c * CHUNK
            pltpu.sync_copy(rbuf, xr_hbm.at[pl.ds(off, CHUNK)])
            pltpu.sync_copy(cbuf, xc_hbm.at[pl.ds(off, CHUNK)])

        issue(0, ridx_a, cidx_a, rbuf_a, cbuf_a, sem_a)

        def loop(p, carry):
            c = 2 * p
            issue(c + 1, ridx_b, cidx_b, rbuf_b, cbuf_b, sem_b)
            drain_store(c, ridx_a, cidx_a, rbuf_a, cbuf_a, sem_a)
            issue(c + 2, ridx_a, cidx_a, rbuf_a, cbuf_a, sem_a)
            drain_store(c + 1, ridx_b, cidx_b, rbuf_b, cbuf_b, sem_b)
            return carry

        lax.fori_loop(0, (NCHUNK_H - 1) // 2, loop, 0)
        drain_store(NCHUNK_H - 1, ridx_a, cidx_a, rbuf_a, cbuf_a, sem_a)

    fn = functools.partial(
        pl.kernel,
        out_type=[jax.ShapeDtypeStruct((e_tot, D), jnp.float32),
                  jax.ShapeDtypeStruct((e_tot, D), jnp.float32)],
        mesh=plsc.VectorSubcoreMesh(core_axis_name="c", subcore_axis_name="s"),
        scratch_types=[
            pltpu.VMEM((CHUNK,), jnp.int32),
            pltpu.VMEM((CHUNK,), jnp.int32),
            pltpu.VMEM((CHUNK, D), jnp.float32),
            pltpu.VMEM((CHUNK, D), jnp.float32),
            pltpu.VMEM((CHUNK,), jnp.int32),
            pltpu.VMEM((CHUNK,), jnp.int32),
            pltpu.VMEM((CHUNK, D), jnp.float32),
            pltpu.VMEM((CHUNK, D), jnp.float32),
            pltpu.SemaphoreType.DMA,
            pltpu.SemaphoreType.DMA,
        ],
    )(body)
    return fn(x, row1, col1)


def _edge_body(xr_ref, xc_ref, ea_ref, em_ref,
               w1r_ref, w1c_ref, w1e_ref, b1_ref, w2_ref, b2_ref, agg_ref):
    gr = xr_ref[...]
    gc = xc_ref[...]

    def tan_scale(g0):
        g0c = jnp.maximum(g0, 1.0 + EPS)
        n = jnp.sqrt((g0c - 1.0) * (g0c + 1.0))
        return jnp.log(g0c + n) / jnp.maximum(n, 1e-8)

    tr = gr * tan_scale(gr[:, 0:1])
    tcv = gc * tan_scale(gc[:, 0:1])
    s = jnp.dot(tr, w1r_ref[...], preferred_element_type=jnp.float32)
    s = s + jnp.dot(tcv, w1c_ref[...], preferred_element_type=jnp.float32)
    s = s + jnp.dot(ea_ref[...], w1e_ref[...], preferred_element_type=jnp.float32)
    s = s + b1_ref[...]
    hid = s / (1.0 + jnp.exp(-s))
    logit = jnp.sum(hid * w2_ref[...], axis=1, keepdims=True) + b2_ref[...][:, 0:1]
    att = em_ref[...] / (1.0 + jnp.exp(-logit))
    alpha = jnp.maximum(
        2.0 * gr[:, 0:1] * gc[:, 0:1] - jnp.sum(gr * gc, axis=1, keepdims=True),
        1.0 + EPS)
    ssq = jnp.sqrt((alpha - 1.0) * (alpha + 1.0))
    dd = jnp.log(alpha + ssq)
    agg_ref[...] = (att * dd / ssq) * (gc - alpha * gr)


def _edge_stage(xr, xc, ea_pad, edge_mask, w1r, w1c, w1e, b1, w2row, b2row):
    e_tot = xr.shape[0]
    grid = (e_tot // BE,)
    return pl.pallas_call(
        _edge_body,
        grid=grid,
        in_specs=[
            pl.BlockSpec((BE, D), lambda i: (i, 0)),
            pl.BlockSpec((BE, D), lambda i: (i, 0)),
            pl.BlockSpec((BE, 8), lambda i: (i, 0)),
            pl.BlockSpec((BE, 1), lambda i: (i, 0)),
            pl.BlockSpec((D, D), lambda i: (0, 0)),
            pl.BlockSpec((D, D), lambda i: (0, 0)),
            pl.BlockSpec((8, D), lambda i: (0, 0)),
            pl.BlockSpec((1, D), lambda i: (0, 0)),
            pl.BlockSpec((1, D), lambda i: (0, 0)),
            pl.BlockSpec((1, D), lambda i: (0, 0)),
        ],
        out_specs=pl.BlockSpec((BE, D), lambda i: (i, 0)),
        out_shape=jax.ShapeDtypeStruct((e_tot, D), jnp.float32),
    )(xr, xc, ea_pad, edge_mask, w1r, w1c, w1e, b1, w2row, b2row)


def _sc_scatter(agg, row1, zeros_tile):
    def body(agg_hbm, row_hbm, z_hbm, out_hbm,
             idx_a, val_a, idx_b, val_b, acc_sh, sem_a, sem_b):
        c = lax.axis_index("c")
        s = lax.axis_index("s")
        wid = s * NC + c
        rbase = s * RPT
        pltpu.sync_copy(z_hbm, acc_sh.at[pl.ds(rbase, RPT)])
        plsc.subcore_barrier()

        def issue(j, idx, val, sem):
            off = wid * EPT_H + j * CHUNK
            pltpu.async_copy(row_hbm.at[pl.ds(off, CHUNK)], idx, sem)
            pltpu.async_copy(agg_hbm.at[pl.ds(off, CHUNK)], val, sem)

        def drain_scatter(j, idx, val, sem):
            off = wid * EPT_H + j * CHUNK
            pltpu.make_async_copy(row_hbm.at[pl.ds(off, CHUNK)], idx, sem).wait()
            pltpu.make_async_copy(agg_hbm.at[pl.ds(off, CHUNK)], val, sem).wait()
            pltpu.sync_copy(val, acc_sh.at[idx], add=True)

        issue(0, idx_a, val_a, sem_a)

        def loop(p, carry):
            j = 2 * p
            issue(j + 1, idx_b, val_b, sem_b)
            drain_scatter(j, idx_a, val_a, sem_a)
            issue(j + 2, idx_a, val_a, sem_a)
            drain_scatter(j + 1, idx_b, val_b, sem_b)
            return carry

        lax.fori_loop(0, (NCHUNK_H - 1) // 2, loop, 0)
        drain_scatter(NCHUNK_H - 1, idx_a, val_a, sem_a)
        plsc.subcore_barrier()
        pltpu.sync_copy(acc_sh.at[pl.ds(rbase, RPT)],
                        out_hbm.at[c, pl.ds(rbase, RPT)])

    fn = functools.partial(
        pl.kernel,
        out_type=jax.ShapeDtypeStruct((NC, NPAD, D), jnp.float32),
        mesh=plsc.VectorSubcoreMesh(core_axis_name="c", subcore_axis_name="s"),
        scratch_types=[
            pltpu.VMEM((CHUNK,), jnp.int32),
            pltpu.VMEM((CHUNK, D), jnp.float32),
            pltpu.VMEM((CHUNK,), jnp.int32),
            pltpu.VMEM((CHUNK, D), jnp.float32),
            pltpu.VMEM_SHARED((NPAD, D), jnp.float32),
            pltpu.SemaphoreType.DMA,
            pltpu.SemaphoreType.DMA,
        ],
    )(body)
    return fn(agg, row1, zeros_tile)


def _final_body(x_ref, p0_ref, p1_ref, p2_ref, p3_ref, gam_ref, bet_ref, out_ref):
    x = x_ref[...]
    sp = _lane_mask(x.shape)
    agg = (p0_ref[...] + p1_ref[...]) + (p2_ref[...] + p3_ref[...])
    li = jnp.sum(jnp.where(sp, x * agg, -(x * agg)), axis=1, keepdims=True)
    u = agg + li * x
    linn = jnp.sum(jnp.where(sp, u * u, -(u * u)), axis=1, keepdims=True)
    nrm = jnp.sqrt(jnp.maximum(linn, 1e-12))
    ch, sh = _cosh_sinh(nrm)
    x2 = ch * x + (sh / nrm) * u
    d = _arccosh(jnp.maximum(x2[:, 0:1], 1.0 + EPS))
    n = jnp.maximum(
        jnp.sqrt(jnp.sum(jnp.where(sp, x2 * x2, 0.0), axis=1, keepdims=True)), 1e-8)
    ht = jnp.where(sp, x2 * (d / n), 0.0)
    mu = jnp.sum(ht, axis=1, keepdims=True) / 127.0
    dsp = jnp.where(sp, ht - mu, 0.0)
    var = jnp.sum(dsp * dsp, axis=1, keepdims=True) / 127.0
    spn = dsp / jnp.sqrt(var + 1e-5) * gam_ref[...] + bet_ref[...]
    n3 = jnp.maximum(
        jnp.sqrt(jnp.sum(jnp.where(sp, spn * spn, 0.0), axis=1, keepdims=True)),
        1e-8)
    c3, s3 = _cosh_sinh(n3)
    x3 = jnp.where(sp, spn * (s3 / n3), c3)
    d4 = _arccosh(jnp.maximum(x3[:, 0:1], 1.0 + EPS))
    n4 = jnp.maximum(
        jnp.sqrt(jnp.sum(jnp.where(sp, x3 * x3, 0.0), axis=1, keepdims=True)), 1e-8)
    r = jnp.maximum(jnp.where(sp, x3 * (d4 / n4), 0.0), 0.0)
    n5 = jnp.maximum(jnp.sqrt(jnp.sum(r * r, axis=1, keepdims=True)), 1e-8)
    c5, s5 = _cosh_sinh(n5)
    out_ref[...] = jnp.where(sp, r * (s5 / n5), c5)


def _final_stage(x, p0, p1, p2, p3, gam, bet):
    return pl.pallas_call(
        _final_body,
        out_shape=jax.ShapeDtypeStruct((N, D), jnp.float32),
    )(x, p0, p1, p2, p3, gam, bet)


def kernel(h, edge_attr, edges, node_mask, edge_mask, W, b, gamma, beta,
           aW1, ab1, aW2, ab2):
    del node_mask
    row = edges[0].astype(jnp.int32)
    col = edges[1].astype(jnp.int32)

    x = _stage_a(h, W, b)

    ea_pad = jnp.concatenate(
        [edge_attr, jnp.zeros((E, 8 - edge_attr.shape[1]), jnp.float32)], axis=1)
    lane0 = jnp.arange(D)[:, None] > 0
    w1r = aW1[:D] * lane0
    w1c = aW1[D:2 * D] * lane0
    w1e = jnp.concatenate(
        [aW1[2 * D:], jnp.zeros((8 - (aW1.shape[0] - 2 * D), D), jnp.float32)],
        axis=0)
    b1 = ab1.reshape(1, D)
    w2row = aW2.reshape(1, D)
    b2row = jnp.broadcast_to(ab2.reshape(1, 1), (1, D))
    zeros_tile = jnp.zeros((RPT, D), jnp.float32)

    parts = []
    for half in range(NSPLIT):
        sl = slice(half * EH, (half + 1) * EH)
        r_h, c_h = row[sl], col[sl]
        xr, xc = _sc_gather(x, r_h, c_h)
        agg = _edge_stage(xr, xc, ea_pad[sl], edge_mask[sl], w1r, w1c, w1e,
                          b1, w2row, b2row)
        parts.append(_sc_scatter(agg, r_h, zeros_tile))

    gam = jnp.concatenate([jnp.ones((1,), jnp.float32), gamma]).reshape(1, D)
    bet = jnp.concatenate([jnp.zeros((1,), jnp.float32), beta]).reshape(1, D)
    return _final_stage(x, parts[0][0, :N], parts[0][1, :N],
                        parts[1][0, :N], parts[1][1, :N], gam, bet)
